# baseline (speedup 1.0000x reference)
import functools

import jax
import jax.numpy as jnp
from jax import lax
from jax.experimental import pallas as pl
from jax.experimental.pallas import tpu as pltpu
from jax.experimental.pallas import tpu_sc as plsc

NUM_EXPERTS = 8
HIDDEN = 1024
INTER = 2048
T = 2048

RB = 512
NB = 15
CAP = NB * RB
FT = 512
NFT = INTER // FT

NC = 2
NS = 16
NW = NC * NS
ROWS_W = CAP // NW
GCH = 48
TCH = T // NW
ECH = 16

def _mesh():
    return plsc.VectorSubcoreMesh(core_axis_name="c", subcore_axis_name="s",
                                  num_cores=NC, num_subcores=NS)


def _route_body(logits_ref, pos0_ref, pos1_ref, w0_ref, w1_ref,
                be_ref, ba_ref, xb_ref):
    E = NUM_EXPERTS
    logits = logits_ref[...].astype(jnp.float32)
    iota_e = lax.broadcasted_iota(jnp.int32, (T, E), 1)
    m0 = jnp.max(logits, axis=1, keepdims=True)
    idx0 = jnp.min(jnp.where(logits == m0, iota_e, E), axis=1, keepdims=True)
    masked = jnp.where(iota_e == idx0, -jnp.inf, logits)
    m1 = jnp.max(masked, axis=1, keepdims=True)
    idx1 = jnp.min(jnp.where(masked == m1, iota_e, E), axis=1, keepdims=True)
    w0 = 1.0 / (1.0 + jnp.exp(m1 - m0))
    oh0 = (iota_e == idx0).astype(jnp.float32)
    oh1 = (iota_e == idx1).astype(jnp.float32)

    def excl_cumsum(a):
        s = a
        sh = 1
        while sh < T:
            s = s + jnp.concatenate(
                [jnp.zeros((sh, E), jnp.float32), s[:T - sh]], axis=0)
            sh *= 2
        return s - a

    c0 = excl_cumsum(oh0)
    c1 = excl_cumsum(oh1)
    count0 = jnp.sum(oh0, axis=0, keepdims=True)
    count = count0 + jnp.sum(oh1, axis=0, keepdims=True)
    pc = jnp.ceil(count / RB) * RB
    tri = (lax.broadcasted_iota(jnp.int32, (E, E), 0)
           < lax.broadcasted_iota(jnp.int32, (E, E), 1)).astype(jnp.float32)
    offs = lax.dot_general(pc, tri, (((1,), (0,)), ((), ())),
                           preferred_element_type=jnp.float32)
    total_used = jnp.sum(pc)
    rank0 = jnp.sum(oh0 * c0, axis=1, keepdims=True)
    rank1 = (jnp.sum(oh1 * c1, axis=1, keepdims=True)
             + jnp.sum(oh1 * count0, axis=1, keepdims=True))
    pos0_ref[...] = (jnp.sum(oh0 * offs, axis=1, keepdims=True)
                     + rank0).astype(jnp.int32)
    pos1_ref[...] = (jnp.sum(oh1 * offs, axis=1, keepdims=True)
                     + rank1).astype(jnp.int32)
    w0_ref[...] = w0
    w1_ref[...] = 1.0 - w0

    iota_b = lax.broadcasted_iota(jnp.int32, (1, NB), 1)
    bb = (iota_b * RB).astype(jnp.float32)
    bbase = jnp.minimum(bb, total_used - 1.0)
    acc = jnp.zeros((1, NB), jnp.float32)
    for e in range(E):
        off_e = lax.slice(offs, (0, e), (1, e + 1))
        acc = acc + (bbase >= off_e).astype(jnp.float32)
    be_ref[...] = (acc - 1.0).astype(jnp.int32)
    ba_ref[...] = (bb < total_used).astype(jnp.int32)
    nbt = (total_used / RB).astype(jnp.int32)
    xb_ref[...] = jnp.minimum(iota_b, nbt - 1)


def _route(router_logits):
    return pl.pallas_call(
        _route_body,
        grid=(1,),
        in_specs=[pl.BlockSpec((T, NUM_EXPERTS), lambda i: (0, 0))],
        out_specs=[
            pl.BlockSpec((T, 1), lambda i: (0, 0)),
            pl.BlockSpec((T, 1), lambda i: (0, 0)),
            pl.BlockSpec((T, 1), lambda i: (0, 0)),
            pl.BlockSpec((T, 1), lambda i: (0, 0)),
            pl.BlockSpec((1, NB), lambda i: (0, 0)),
            pl.BlockSpec((1, NB), lambda i: (0, 0)),
            pl.BlockSpec((1, NB), lambda i: (0, 0)),
        ],
        out_shape=[
            jax.ShapeDtypeStruct((T, 1), jnp.int32),
            jax.ShapeDtypeStruct((T, 1), jnp.int32),
            jax.ShapeDtypeStruct((T, 1), jnp.float32),
            jax.ShapeDtypeStruct((T, 1), jnp.float32),
            jax.ShapeDtypeStruct((1, NB), jnp.int32),
            jax.ShapeDtypeStruct((1, NB), jnp.int32),
            jax.ShapeDtypeStruct((1, NB), jnp.int32),
        ],
    )(router_logits)


def _dispatch_body(pos0_hbm, pos1_hbm, w0_hbm, w1_hbm, x_hbm,
              xs_hbm, rs_hbm,
              pos0_v, pos1_v, w0_v, w1_v, rt_v, rs_v, idx_v, rows_v,
              rt_sh, sem):
    c = lax.axis_index("c")
    s = lax.axis_index("s")

    @pl.when(s == 0)
    def _():
        pltpu.sync_copy(pos0_hbm, pos0_v)
        pltpu.sync_copy(pos1_hbm, pos1_v)
        pltpu.sync_copy(w0_hbm, w0_v)
        pltpu.sync_copy(w1_hbm, w1_v)

        def zero_body(j, carry):
            rt_v[pl.ds(j * 16, 16)] = jnp.zeros((16,), jnp.int32)
            rs_v[pl.ds(j * 16, 16)] = jnp.zeros((16,), jnp.float32)
            return carry

        lax.fori_loop(0, CAP // 16, zero_body, 0)
        lane = lax.iota(jnp.int32, 16)

        def scat_body(j, carry):
            tok = lane + j * 16
            i0 = pos0_v[pl.ds(j * 16, 16)]
            plsc.store_scatter(rt_v, [i0], tok)
            plsc.store_scatter(rs_v, [i0], w0_v[pl.ds(j * 16, 16)])
            i1 = pos1_v[pl.ds(j * 16, 16)]
            plsc.store_scatter(rt_v, [i1], tok)
            plsc.store_scatter(rs_v, [i1], w1_v[pl.ds(j * 16, 16)])
            return carry

        lax.fori_loop(0, T // 16, scat_body, 0)
        pltpu.sync_copy(rt_v, rt_sh)

        @pl.when(c == 0)
        def _():
            pltpu.sync_copy(rs_v, rs_hbm)

    plsc.subcore_barrier()
    w = s * NC + c
    base = w * ROWS_W

    def gather_body(i, carry):
        off = base + i * GCH
        pltpu.sync_copy(rt_sh.at[pl.ds(off, GCH)], idx_v)
        pltpu.async_copy(x_hbm.at[idx_v], rows_v, sem).wait()
        pltpu.sync_copy(rows_v, xs_hbm.at[pl.ds(off, GCH)])
        return carry

    lax.fori_loop(0, ROWS_W // GCH, gather_body, 0)


_dispatch_impl = None


def _dispatch(pos0f, pos1f, w0f, w1f, x):
    global _dispatch_impl
    if _dispatch_impl is None:
        _dispatch_impl = pl.kernel(
            _dispatch_body,
            out_type=[
                jax.ShapeDtypeStruct((CAP, HIDDEN), jnp.float32),
                jax.ShapeDtypeStruct((CAP,), jnp.float32),
            ],
            mesh=_mesh(),
            scratch_types=[
                pltpu.VMEM((T,), jnp.int32),
                pltpu.VMEM((T,), jnp.int32),
                pltpu.VMEM((T,), jnp.float32),
                pltpu.VMEM((T,), jnp.float32),
                pltpu.VMEM((CAP,), jnp.int32),
                pltpu.VMEM((CAP,), jnp.float32),
                pltpu.VMEM((GCH,), jnp.int32),
                pltpu.VMEM((GCH, HIDDEN), jnp.float32),
                pltpu.VMEM_SHARED((CAP,), jnp.int32),
                pltpu.SemaphoreType.DMA,
            ],
            compiler_params=pltpu.CompilerParams(needs_layout_passes=False),
        )
    return _dispatch_impl(pos0f, pos1f, w0f, w1f, x)


def _gmm_body(be_ref, ba_ref, xb_ref,
              xs_ref, rs_ref, w13g_ref, w13u_ref, w2_ref, y_ref):
    b = pl.program_id(0)
    f = pl.program_id(1)

    @pl.when(ba_ref[b] == 1)
    def _():
        x = xs_ref[...].astype(jnp.bfloat16)
        gate = lax.dot_general(x, w13g_ref[0], (((1,), (1,)), ((), ())),
                               preferred_element_type=jnp.float32)
        up = lax.dot_general(x, w13u_ref[0], (((1,), (1,)), ((), ())),
                             preferred_element_type=jnp.float32)
        act = (gate * jax.nn.sigmoid(gate) * up).astype(jnp.bfloat16)
        part = lax.dot_general(act, w2_ref[0], (((1,), (1,)), ((), ())),
                               preferred_element_type=jnp.float32)

        @pl.when(f == 0)
        def _():
            y_ref[...] = part

        @pl.when(f > 0)
        def _():
            y_ref[...] += part

        @pl.when(f == NFT - 1)
        def _():
            y_ref[...] *= rs_ref[...]


def _feff(ba_ref, b, f):
    return jnp.where(ba_ref[b] == 0, NFT - 1, f)


def _gmm(bev, bav, xbv, xs, rs, w13_16, w2_16):
    grid_spec = pltpu.PrefetchScalarGridSpec(
        num_scalar_prefetch=3,
        grid=(NB, NFT),
        in_specs=[
            pl.BlockSpec((RB, HIDDEN), lambda b, f, be, ba, xb: (xb[b], 0)),
            pl.BlockSpec((RB, 1), lambda b, f, be, ba, xb: (xb[b], 0)),
            pl.BlockSpec((1, FT, HIDDEN),
                         lambda b, f, be, ba, xb: (be[b], _feff(ba, b, f), 0)),
            pl.BlockSpec((1, FT, HIDDEN),
                         lambda b, f, be, ba, xb:
                         (be[b], NFT + _feff(ba, b, f), 0)),
            pl.BlockSpec((1, HIDDEN, FT),
                         lambda b, f, be, ba, xb: (be[b], 0, _feff(ba, b, f))),
        ],
        out_specs=pl.BlockSpec((RB, HIDDEN), lambda b, f, be, ba, xb: (b, 0)),
    )
    return pl.pallas_call(
        _gmm_body,
        grid_spec=grid_spec,
        out_shape=jax.ShapeDtypeStruct((CAP, HIDDEN), jnp.float32),
        compiler_params=pltpu.CompilerParams(
            dimension_semantics=("arbitrary", "arbitrary"),
        ),
    )(bev, bav, xbv, xs, rs, w13_16, w13_16, w2_16)


def _combine_body(pos0_hbm, pos1_hbm, y_hbm, out_hbm,
             i0_v, i1_v, r0_v, r1_v, s0, s1):
    c = lax.axis_index("c")
    s = lax.axis_index("s")
    w = s * NC + c
    tb = w * TCH
    pltpu.sync_copy(pos0_hbm.at[pl.ds(tb, TCH)], i0_v)
    pltpu.sync_copy(pos1_hbm.at[pl.ds(tb, TCH)], i1_v)

    def chunk_body(ci, carry):
        cp0 = pltpu.async_copy(
            y_hbm.at[i0_v.at[pl.ds(ci * ECH, ECH)]], r0_v, s0)
        cp1 = pltpu.async_copy(
            y_hbm.at[i1_v.at[pl.ds(ci * ECH, ECH)]], r1_v, s1)
        cp0.wait()
        cp1.wait()

        def row_body(i, carry2):
            def col_body(j, carry3):
                r0_v[i, pl.ds(j * 16, 16)] = (
                    r0_v[i, pl.ds(j * 16, 16)] + r1_v[i, pl.ds(j * 16, 16)])
                return carry3
            lax.fori_loop(0, HIDDEN // 16, col_body, 0)
            return carry2

        lax.fori_loop(0, ECH, row_body, 0)
        pltpu.sync_copy(r0_v, out_hbm.at[pl.ds(tb + ci * ECH, ECH)])
        return carry

    lax.fori_loop(0, TCH // ECH, chunk_body, 0)


_combine_impl = None


def _combine(pos0f, pos1f, y):
    global _combine_impl
    if _combine_impl is None:
        _combine_impl = pl.kernel(
            _combine_body,
            out_type=jax.ShapeDtypeStruct((T, HIDDEN), jnp.float32),
            mesh=_mesh(),
            scratch_types=[
                pltpu.VMEM((TCH,), jnp.int32),
                pltpu.VMEM((TCH,), jnp.int32),
                pltpu.VMEM((ECH, HIDDEN), jnp.float32),
                pltpu.VMEM((ECH, HIDDEN), jnp.float32),
                pltpu.SemaphoreType.DMA,
                pltpu.SemaphoreType.DMA,
            ],
            compiler_params=pltpu.CompilerParams(needs_layout_passes=False),
        )
    return _combine_impl(pos0f, pos1f, y)


def kernel(hidden_states, router_logits, w13_weight, w2_weight):
    w13_16 = w13_weight.astype(jnp.bfloat16)
    w2_16 = w2_weight.astype(jnp.bfloat16)

    pos0, pos1, w0, w1, be, ba, xb = _route(router_logits)
    pos0f = pos0.reshape(T)
    pos1f = pos1.reshape(T)

    xs, rs = _dispatch(pos0f, pos1f, w0.reshape(T), w1.reshape(T),
                       hidden_states)
    y = _gmm(be.reshape(NB), ba.reshape(NB), xb.reshape(NB),
             xs, rs.reshape(CAP, 1), w13_16, w2_16)
    return _combine(pos0f, pos1f, y)
